# baseline (speedup 1.0000x reference)
import functools

import jax
import jax.numpy as jnp
from jax import lax
from jax.experimental import pallas as pl
from jax.experimental.pallas import tpu as pltpu
from jax.experimental.pallas import tpu_sc as plsc

BATCH = 16384
EMBED_DIM = 16

_info = plsc.get_sparse_core_info()
_NC, _NS = _info.num_cores, _info.num_subcores
_NW = _NC * _NS
_B_PER_W = BATCH // _NW


@jax.jit
def kernel(labels, table):
    mesh = plsc.VectorSubcoreMesh(core_axis_name="c", subcore_axis_name="s")

    @functools.partial(
        pl.kernel,
        mesh=mesh,
        out_type=jax.ShapeDtypeStruct((BATCH, EMBED_DIM), jnp.float32),
        scratch_types=[
            pltpu.VMEM((_B_PER_W,), jnp.int32),
            pltpu.VMEM((_B_PER_W, EMBED_DIM), jnp.float32),
            pltpu.SemaphoreType.DMA,
        ],
        compiler_params=pltpu.CompilerParams(use_tc_tiling_on_sc=False),
    )
    def _gather(labels_hbm, table_hbm, out_hbm, idx_v, rows_v, sem):
        wid = lax.axis_index("s") * _NC + lax.axis_index("c")
        base = wid * _B_PER_W
        pltpu.sync_copy(labels_hbm.at[pl.ds(base, _B_PER_W)], idx_v)
        pltpu.async_copy(table_hbm.at[idx_v], rows_v, sem).wait()
        pltpu.sync_copy(rows_v, out_hbm.at[pl.ds(base, _B_PER_W)])

    return _gather(labels.astype(jnp.int32), table)


# speedup vs baseline: 5.9237x; 5.9237x over previous
import functools

import jax
import jax.numpy as jnp
from jax import lax
from jax.experimental import pallas as pl
from jax.experimental.pallas import tpu as pltpu
from jax.experimental.pallas import tpu_sc as plsc

BATCH = 16384
EMBED_DIM = 16
NUM_ROWS = 1000001

_info = plsc.get_sparse_core_info()
_NC, _NS = _info.num_cores, _info.num_subcores
_NW = _NC * _NS
_B_PER_W = BATCH // _NW
_G = 16
_NGROUPS = _B_PER_W // _G


@jax.jit
def kernel(labels, table):
    mesh = plsc.VectorSubcoreMesh(core_axis_name="c", subcore_axis_name="s")

    @functools.partial(
        pl.kernel,
        mesh=mesh,
        out_type=jax.ShapeDtypeStruct((EMBED_DIM, BATCH), jnp.float32),
        scratch_types=[
            pltpu.VMEM((_B_PER_W,), jnp.int32),
            pltpu.VMEM((EMBED_DIM, _B_PER_W), jnp.float32),
            pltpu.VMEM((_G, EMBED_DIM, 128), jnp.float32),
            pltpu.VMEM((_G, EMBED_DIM, 128), jnp.float32),
            pltpu.SemaphoreType.DMA,
            pltpu.SemaphoreType.DMA,
        ],
        compiler_params=pltpu.CompilerParams(
            use_tc_tiling_on_sc=True, needs_layout_passes=False
        ),
    )
    def _gather(labels_hbm, table_t_hbm, out_hbm, idx_v, rows_v, buf0, buf1,
                sem0, sem1):
        wid = lax.axis_index("s") * _NC + lax.axis_index("c")
        base = wid * _B_PER_W
        pltpu.sync_copy(labels_hbm.at[pl.ds(base, _B_PER_W)], idx_v)

        bufs = (buf0, buf1)
        sems = (sem0, sem1)
        lane_iota = lax.iota(jnp.int32, 16)

        def issue_group(g, buf, sem):
            vec = idx_v[pl.ds(g * _G, _G)]
            for j in range(_G):
                r = jnp.squeeze(lax.slice(vec, (j,), (j + 1,)))
                start = pl.multiple_of((r // 128) * 128, 128)
                pltpu.async_copy(
                    table_t_hbm.at[:, pl.ds(start, 128)],
                    buf.at[j],
                    sem,
                )

        def drain_group(buf, sem):
            for j in range(_G):
                pltpu.make_async_copy(
                    table_t_hbm.at[:, pl.ds(0, 128)], buf.at[j], sem
                ).wait()

        def process_group(g, buf):
            vec = idx_v[pl.ds(g * _G, _G)]
            for j in range(_G):
                r = jnp.squeeze(lax.slice(vec, (j,), (j + 1,)))
                lane = jnp.full((16,), r % 128, jnp.int32)
                row = plsc.load_gather(buf.at[j], [lane_iota, lane])
                col = jnp.full((16,), g * _G + j, jnp.int32)
                plsc.store_scatter(rows_v, [lane_iota, col], row)

        issue_group(0, bufs[0], sems[0])

        def body(i):
            g = i * 2
            issue_group(g + 1, bufs[1], sems[1])
            drain_group(bufs[0], sems[0])
            process_group(g, bufs[0])

            @pl.when(g + 2 < _NGROUPS)
            def _():
                issue_group(g + 2, bufs[0], sems[0])

            drain_group(bufs[1], sems[1])
            process_group(g + 1, bufs[1])

        pl.loop(0, _NGROUPS // 2)(body)
        pltpu.sync_copy(rows_v, out_hbm.at[:, pl.ds(base, _B_PER_W)])

    out_t = _gather(labels.astype(jnp.int32), table.T)
    return out_t.T


# speedup vs baseline: 6.0857x; 1.0273x over previous
import functools

import jax
import jax.numpy as jnp
from jax import lax
from jax.experimental import pallas as pl
from jax.experimental.pallas import tpu as pltpu
from jax.experimental.pallas import tpu_sc as plsc

BATCH = 16384
EMBED_DIM = 16
NUM_ROWS = 1000001

_info = plsc.get_sparse_core_info()
_NC, _NS = _info.num_cores, _info.num_subcores
_NW = _NC * _NS
_B_PER_W = BATCH // _NW
_G = 16
_NGROUPS = _B_PER_W // _G


@jax.jit
def kernel(labels, table):
    mesh = plsc.VectorSubcoreMesh(core_axis_name="c", subcore_axis_name="s")

    @functools.partial(
        pl.kernel,
        mesh=mesh,
        out_type=jax.ShapeDtypeStruct((EMBED_DIM, BATCH), jnp.float32),
        scratch_types=[
            pltpu.VMEM((_B_PER_W,), jnp.int32),
            pltpu.VMEM((EMBED_DIM, _B_PER_W), jnp.float32),
            pltpu.VMEM((EMBED_DIM, _G * 128), jnp.float32),
            pltpu.VMEM((EMBED_DIM, _G * 128), jnp.float32),
            pltpu.SemaphoreType.DMA,
            pltpu.SemaphoreType.DMA,
        ],
        compiler_params=pltpu.CompilerParams(
            use_tc_tiling_on_sc=True, needs_layout_passes=False
        ),
    )
    def _gather(labels_hbm, table_t_hbm, out_hbm, idx_v, rows_v, buf0, buf1,
                sem0, sem1):
        wid = lax.axis_index("s") * _NC + lax.axis_index("c")
        base = wid * _B_PER_W
        pltpu.sync_copy(labels_hbm.at[pl.ds(base, _B_PER_W)], idx_v)

        bufs = (buf0, buf1)
        sems = (sem0, sem1)
        lane_iota = lax.iota(jnp.int32, 16)

        def issue_group(g, buf, sem):
            starts = (idx_v[pl.ds(g * _G, _G)] // 128) * 128
            for j in range(_G):
                start = pl.multiple_of(
                    jnp.squeeze(lax.slice(starts, (j,), (j + 1,))), 128
                )
                pltpu.async_copy(
                    table_t_hbm.at[:, pl.ds(start, 128)],
                    buf.at[:, pl.ds(j * 128, 128)],
                    sem,
                )

        def drain_group(buf, sem):
            pltpu.make_async_copy(
                table_t_hbm.at[:, pl.ds(0, _G * 128)], buf, sem
            ).wait()

        def process_group(g, buf):
            cols = idx_v[pl.ds(g * _G, _G)] % 128 + 128 * lane_iota
            for c in range(EMBED_DIM):
                plane = jnp.full((16,), c, jnp.int32)
                row = plsc.load_gather(buf, [plane, cols])
                plsc.store_scatter(
                    rows_v, [plane, g * _G + lane_iota], row
                )

        issue_group(0, bufs[0], sems[0])

        def body(i):
            g = i * 2
            issue_group(g + 1, bufs[1], sems[1])
            drain_group(bufs[0], sems[0])
            process_group(g, bufs[0])

            @pl.when(g + 2 < _NGROUPS)
            def _():
                issue_group(g + 2, bufs[0], sems[0])

            drain_group(bufs[1], sems[1])
            process_group(g + 1, bufs[1])

        pl.loop(0, _NGROUPS // 2)(body)
        pltpu.sync_copy(rows_v, out_hbm.at[:, pl.ds(base, _B_PER_W)])

    out_t = _gather(labels.astype(jnp.int32), table.T)
    return out_t.T


# speedup vs baseline: 6.6383x; 1.0908x over previous
import functools

import jax
import jax.numpy as jnp
from jax import lax
from jax.experimental import pallas as pl
from jax.experimental.pallas import tpu as pltpu
from jax.experimental.pallas import tpu_sc as plsc

BATCH = 16384
EMBED_DIM = 16
NUM_ROWS = 1000001

_info = plsc.get_sparse_core_info()
_NC, _NS = _info.num_cores, _info.num_subcores
_NW = _NC * _NS
_B_PER_W = BATCH // _NW
_G = 16
_NGROUPS = _B_PER_W // _G


@jax.jit
def kernel(labels, table):
    mesh = plsc.VectorSubcoreMesh(core_axis_name="c", subcore_axis_name="s")

    @functools.partial(
        pl.kernel,
        mesh=mesh,
        out_type=jax.ShapeDtypeStruct((EMBED_DIM, BATCH), jnp.float32),
        scratch_types=[
            pltpu.VMEM((_B_PER_W,), jnp.int32),
            pltpu.VMEM((EMBED_DIM, _B_PER_W), jnp.float32),
            pltpu.VMEM((EMBED_DIM, _G * 128), jnp.float32),
            pltpu.VMEM((EMBED_DIM, _G * 128), jnp.float32),
            pltpu.VMEM((EMBED_DIM, _G * 128), jnp.float32),
            pltpu.SemaphoreType.DMA,
            pltpu.SemaphoreType.DMA,
            pltpu.SemaphoreType.DMA,
        ],
        compiler_params=pltpu.CompilerParams(
            use_tc_tiling_on_sc=True, needs_layout_passes=False
        ),
    )
    def _gather(labels_hbm, table_t_hbm, out_hbm, idx_v, rows_v, buf0, buf1,
                buf2, sem0, sem1, sem2):
        wid = lax.axis_index("s") * _NC + lax.axis_index("c")
        base = wid * _B_PER_W
        pltpu.sync_copy(labels_hbm.at[pl.ds(base, _B_PER_W)], idx_v)

        bufs = (buf0, buf1, buf2)
        sems = (sem0, sem1, sem2)
        lane_iota = lax.iota(jnp.int32, 16)

        def issue_group(g, buf, sem):
            starts = (idx_v[pl.ds(g * _G, _G)] // 128) * 128
            for j in range(_G):
                start = pl.multiple_of(
                    jnp.squeeze(lax.slice(starts, (j,), (j + 1,))), 128
                )
                pltpu.async_copy(
                    table_t_hbm.at[:, pl.ds(start, 128)],
                    buf.at[:, pl.ds(j * 128, 128)],
                    sem,
                )

        def drain_group(buf, sem):
            pltpu.make_async_copy(
                table_t_hbm.at[:, pl.ds(0, _G * 128)], buf, sem
            ).wait()

        def process_group(g, buf):
            cols = idx_v[pl.ds(g * _G, _G)] % 128 + 128 * lane_iota
            for c in range(EMBED_DIM):
                plane = jnp.full((16,), c, jnp.int32)
                row = plsc.load_gather(buf, [plane, cols])
                plsc.store_scatter(
                    rows_v, [plane, g * _G + lane_iota], row
                )

        issue_group(0, bufs[0], sems[0])
        issue_group(1, bufs[1], sems[1])

        def body(i):
            g = i * 3
            for k in range(3):
                issue_group(g + k + 2, bufs[(k + 2) % 3], sems[(k + 2) % 3])
                drain_group(bufs[k], sems[k])
                process_group(g + k, bufs[k])

        pl.loop(0, (_NGROUPS - 2) // 3)(body)
        drain_group(bufs[0], sems[0])
        process_group(_NGROUPS - 2, bufs[0])
        drain_group(bufs[1], sems[1])
        process_group(_NGROUPS - 1, bufs[1])
        pltpu.sync_copy(rows_v, out_hbm.at[:, pl.ds(base, _B_PER_W)])

    out_t = _gather(labels.astype(jnp.int32), table.T)
    return out_t.T


# speedup vs baseline: 6.6524x; 1.0021x over previous
import functools

import jax
import jax.numpy as jnp
from jax import lax
from jax.experimental import pallas as pl
from jax.experimental.pallas import tpu as pltpu
from jax.experimental.pallas import tpu_sc as plsc

BATCH = 16384
EMBED_DIM = 16
NUM_ROWS = 1000001

_info = plsc.get_sparse_core_info()
_NC, _NS = _info.num_cores, _info.num_subcores
_NW = _NC * _NS
_B_PER_W = BATCH // _NW
_G = 16
_NGROUPS = _B_PER_W // _G


@jax.jit
def kernel(labels, table):
    mesh = plsc.VectorSubcoreMesh(core_axis_name="c", subcore_axis_name="s")

    @functools.partial(
        pl.kernel,
        mesh=mesh,
        out_type=jax.ShapeDtypeStruct((EMBED_DIM, BATCH), jnp.float32),
        scratch_types=[
            pltpu.VMEM((_B_PER_W,), jnp.int32),
            pltpu.VMEM((EMBED_DIM, _B_PER_W), jnp.float32),
            pltpu.VMEM((EMBED_DIM, _G * 128), jnp.float32),
            pltpu.VMEM((EMBED_DIM, _G * 128), jnp.float32),
            pltpu.VMEM((EMBED_DIM, _G * 128), jnp.float32),
            pltpu.SemaphoreType.DMA,
            pltpu.SemaphoreType.DMA,
            pltpu.SemaphoreType.DMA,
        ],
        compiler_params=pltpu.CompilerParams(
            use_tc_tiling_on_sc=True, needs_layout_passes=False
        ),
    )
    def _gather(labels_hbm, table_t_hbm, out_hbm, idx_v, rows_v, buf0, buf1,
                buf2, sem0, sem1, sem2):
        wid = lax.axis_index("s") * _NC + lax.axis_index("c")
        base = wid * _B_PER_W
        pltpu.sync_copy(labels_hbm.at[pl.ds(base, _B_PER_W)], idx_v)

        bufs = (buf0, buf1, buf2)
        sems = (sem0, sem1, sem2)
        lane_iota = lax.iota(jnp.int32, 16)

        def issue_group(g, buf, sem):
            starts = (idx_v[pl.ds(g * _G, _G)] // 128) * 128
            for j in range(_G):
                start = pl.multiple_of(
                    jnp.squeeze(lax.slice(starts, (j,), (j + 1,))), 128
                )
                pltpu.async_copy(
                    table_t_hbm.at[pl.ds(0, 8), pl.ds(start, 128)],
                    buf.at[pl.ds(0, 8), pl.ds(j * 128, 128)],
                    sem,
                )
                pltpu.async_copy(
                    table_t_hbm.at[pl.ds(8, 8), pl.ds(start, 128)],
                    buf.at[pl.ds(8, 8), pl.ds(j * 128, 128)],
                    sem,
                )

        def drain_group(buf, sem):
            pltpu.make_async_copy(
                table_t_hbm.at[:, pl.ds(0, _G * 128)], buf, sem
            ).wait()

        def process_group(g, buf):
            cols = idx_v[pl.ds(g * _G, _G)] % 128 + 128 * lane_iota
            for c in range(EMBED_DIM):
                plane = jnp.full((16,), c, jnp.int32)
                row = plsc.load_gather(buf, [plane, cols])
                plsc.store_scatter(
                    rows_v, [plane, g * _G + lane_iota], row
                )

        issue_group(0, bufs[0], sems[0])
        issue_group(1, bufs[1], sems[1])

        def body(i):
            g = i * 3
            for k in range(3):
                issue_group(g + k + 2, bufs[(k + 2) % 3], sems[(k + 2) % 3])
                drain_group(bufs[k], sems[k])
                process_group(g + k, bufs[k])

        pl.loop(0, (_NGROUPS - 2) // 3)(body)
        drain_group(bufs[0], sems[0])
        process_group(_NGROUPS - 2, bufs[0])
        drain_group(bufs[1], sems[1])
        process_group(_NGROUPS - 1, bufs[1])
        pltpu.sync_copy(rows_v, out_hbm.at[:, pl.ds(base, _B_PER_W)])

    out_t = _gather(labels.astype(jnp.int32), table.T)
    return out_t.T


# speedup vs baseline: 6.6679x; 1.0023x over previous
import functools

import jax
import jax.numpy as jnp
from jax import lax
from jax.experimental import pallas as pl
from jax.experimental.pallas import tpu as pltpu
from jax.experimental.pallas import tpu_sc as plsc

BATCH = 16384
EMBED_DIM = 16
NUM_ROWS = 1000001

_info = plsc.get_sparse_core_info()
_NC, _NS = _info.num_cores, _info.num_subcores
_NW = _NC * _NS
_B_PER_W = BATCH // _NW
_G = 16
_NGROUPS = _B_PER_W // _G


@jax.jit
def kernel(labels, table):
    mesh = plsc.VectorSubcoreMesh(core_axis_name="c", subcore_axis_name="s")

    @functools.partial(
        pl.kernel,
        mesh=mesh,
        out_type=jax.ShapeDtypeStruct((EMBED_DIM, BATCH), jnp.float32),
        scratch_types=[
            pltpu.VMEM((_B_PER_W,), jnp.int32),
            pltpu.VMEM((EMBED_DIM, _B_PER_W), jnp.float32),
            pltpu.VMEM((EMBED_DIM, _G * 128), jnp.float32),
            pltpu.VMEM((EMBED_DIM, _G * 128), jnp.float32),
            pltpu.VMEM((EMBED_DIM, _G * 128), jnp.float32),
            pltpu.SemaphoreType.DMA,
            pltpu.SemaphoreType.DMA,
            pltpu.SemaphoreType.DMA,
        ],
        compiler_params=pltpu.CompilerParams(
            use_tc_tiling_on_sc=True, needs_layout_passes=False
        ),
    )
    def _gather(labels_hbm, table_t_hbm, out_hbm, idx_v, rows_v, buf0, buf1,
                buf2, sem0, sem1, sem2):
        wid = lax.axis_index("s") * _NC + lax.axis_index("c")
        base = wid * _B_PER_W
        pltpu.sync_copy(labels_hbm.at[pl.ds(base, _B_PER_W)], idx_v)

        bufs = (buf0, buf1, buf2)
        sems = (sem0, sem1, sem2)
        lane_iota = lax.iota(jnp.int32, 16)

        def issue_group(g, buf, sem):
            starts = (idx_v[pl.ds(g * _G, _G)] // 128) * 128
            for j in range(_G):
                start = pl.multiple_of(
                    jnp.squeeze(lax.slice(starts, (j,), (j + 1,))), 128
                )
                pltpu.async_copy(
                    table_t_hbm.at[:, pl.ds(start, 128)],
                    buf.at[:, pl.ds(j * 128, 128)],
                    sem,
                )

        def drain_group(buf, sem):
            pltpu.make_async_copy(
                table_t_hbm.at[:, pl.ds(0, _G * 128)], buf, sem
            ).wait()

        def process_group(g, buf):
            cols = idx_v[pl.ds(g * _G, _G)] % 128 + 128 * lane_iota
            for c in range(EMBED_DIM):
                plane = jnp.full((16,), c, jnp.int32)
                row = plsc.load_gather(buf, [plane, cols])
                plsc.store_scatter(
                    rows_v, [plane, g * _G + lane_iota], row
                )

        issue_group(0, bufs[0], sems[0])
        issue_group(1, bufs[1], sems[1])

        def body(i):
            g = i * 3
            for k in range(3):
                issue_group(g + k + 2, bufs[(k + 2) % 3], sems[(k + 2) % 3])
                drain_group(bufs[k], sems[k])
                process_group(g + k, bufs[k])

        pl.loop(0, (_NGROUPS - 2) // 3)(body)
        drain_group(bufs[0], sems[0])
        process_group(_NGROUPS - 2, bufs[0])
        drain_group(bufs[1], sems[1])
        process_group(_NGROUPS - 1, bufs[1])
        pltpu.sync_copy(rows_v, out_hbm.at[:, pl.ds(base, _B_PER_W)])

    out_t = _gather(labels.astype(jnp.int32), table.T)
    return out_t.T
